# baseline (speedup 1.0000x reference)
import jax
import jax.numpy as jnp
from jax import lax
from jax.experimental import pallas as pl
from jax.experimental.pallas import tpu as pltpu
from jax.experimental.pallas import tpu_sc as plsc

N = 10000
NP = 10240
D = 128
E = 320000
NSEED = 3000
NSEED_PAD = 3072

NC = 2
NS = 16

ROWS_PER_TILE = NP // NS
EDGES_PER_TILE = E // NS
CHUNK = 1000
NCHUNK = EDGES_PER_TILE // CHUNK
DEGW = 16
SEEDS_PER_TILE = 2 * NSEED_PAD // (NC * NS)

PCHUNK = 128
E_PAD = 327680
PADE = E_PAD - E
IDX_ROWS = 2 * E_PAD // PCHUNK
CROWS = IDX_ROWS // NC
TROWS = CROWS // NS
IB = 8
NBLK = TROWS // IB

_sc_mesh = plsc.VectorSubcoreMesh(core_axis_name="c", subcore_axis_name="s")


def _deg_body(dst_hbm, zeros_hbm, ones_hbm, out_hbm, acc, ones_v, idx_v):
    c = lax.axis_index("c")
    s = lax.axis_index("s")
    r0 = s * ROWS_PER_TILE

    pltpu.sync_copy(ones_hbm, ones_v)
    pltpu.sync_copy(zeros_hbm.at[pl.ds(r0, ROWS_PER_TILE)],
                    acc.at[pl.ds(r0, ROWS_PER_TILE)])
    plsc.subcore_barrier()

    def body(k, carry):
        base = pl.multiple_of(c * E + s * EDGES_PER_TILE + k * CHUNK, 8)
        pltpu.sync_copy(dst_hbm.at[pl.ds(base, CHUNK)], idx_v)
        pltpu.sync_copy(ones_v, acc.at[idx_v], add=True)
        return carry

    lax.fori_loop(0, NCHUNK, body, 0)
    plsc.subcore_barrier()
    pltpu.sync_copy(acc.at[pl.ds(r0, ROWS_PER_TILE)],
                    out_hbm.at[pl.ds(c * NP + r0, ROWS_PER_TILE)])


@jax.jit
def _deg_call(dst_all):
    zeros = jnp.zeros((NP, DEGW), jnp.float32)
    ones = jnp.ones((CHUNK, DEGW), jnp.float32)
    return pl.kernel(
        _deg_body,
        out_type=[jax.ShapeDtypeStruct((2 * NP, DEGW), jnp.float32)],
        mesh=_sc_mesh,
        compiler_params=pltpu.CompilerParams(use_tc_tiling_on_sc=False),
        scratch_types=[
            pltpu.VMEM_SHARED((NP, DEGW), jnp.float32),
            pltpu.VMEM((CHUNK, DEGW), jnp.float32),
            pltpu.VMEM((CHUNK,), jnp.int32),
        ],
    )(dst_all, zeros, ones)[0]


def _prop_body(u_hbm, src_hbm, dst_hbm, out_hbm,
               acc, rows_a, rows_b, src_i, dst_i, sem_a, sem_b,
               ssem_a, ssem_b):
    c = lax.axis_index("c")
    s = lax.axis_index("s")
    r0 = s * ROWS_PER_TILE

    pltpu.sync_copy(u_hbm.at[pl.ds(c * NP + r0, ROWS_PER_TILE)],
                    acc.at[pl.ds(r0, ROWS_PER_TILE)])
    plsc.subcore_barrier()

    rows = (rows_a, rows_b)
    sems = (sem_a, sem_b)
    ssems = (ssem_a, ssem_b)
    row0 = c * CROWS + s * TROWS

    def outer(k, carry):
        base = pl.multiple_of(row0 + k * IB, 8)
        pltpu.sync_copy(src_hbm.at[pl.ds(base, IB)], src_i)
        pltpu.sync_copy(dst_hbm.at[pl.ds(base, IB)], dst_i)
        g = [pltpu.async_copy(u_hbm.at[src_i.at[0]], rows[0], sems[0]),
             None]
        sc = [None, None]
        for j in range(IB):
            b = j % 2
            nb = (j + 1) % 2
            g[b].wait()
            sc[b] = pltpu.async_copy(rows[b], acc.at[dst_i.at[j]], ssems[b],
                                     add=True)
            if j + 1 < IB:
                if sc[nb] is not None:
                    sc[nb].wait()
                g[nb] = pltpu.async_copy(
                    u_hbm.at[src_i.at[j + 1]], rows[nb], sems[nb])
        sc[0].wait()
        sc[1].wait()
        return carry

    lax.fori_loop(0, NBLK, outer, 0)
    plsc.subcore_barrier()
    pltpu.sync_copy(acc.at[pl.ds(r0, ROWS_PER_TILE)],
                    out_hbm.at[pl.ds(c * NP + r0, ROWS_PER_TILE)])


@jax.jit
def _prop_call(u_all, src2d, dst2d):
    return pl.kernel(
        _prop_body,
        out_type=[jax.ShapeDtypeStruct((2 * NP, D), jnp.float32)],
        mesh=_sc_mesh,
        scratch_types=[
            pltpu.VMEM_SHARED((NP, D), jnp.float32),
            pltpu.VMEM((PCHUNK, D), jnp.float32),
            pltpu.VMEM((PCHUNK, D), jnp.float32),
            pltpu.VMEM((IB, PCHUNK), jnp.int32),
            pltpu.VMEM((IB, PCHUNK), jnp.int32),
            pltpu.SemaphoreType.DMA,
            pltpu.SemaphoreType.DMA,
            pltpu.SemaphoreType.DMA,
            pltpu.SemaphoreType.DMA,
        ],
    )(u_all, src2d, dst2d)[0]


def _seed_body(ent_hbm, seeds_hbm, out_hbm, idx_v, rows_v, gsem):
    c = lax.axis_index("c")
    s = lax.axis_index("s")
    base = (c * NS + s) * SEEDS_PER_TILE

    pltpu.sync_copy(seeds_hbm.at[pl.ds(base, SEEDS_PER_TILE)], idx_v)
    pltpu.async_copy(ent_hbm.at[idx_v], rows_v, gsem).wait()
    pltpu.sync_copy(rows_v, out_hbm.at[pl.ds(base, SEEDS_PER_TILE)])


@jax.jit
def _seed_call(ent_all, seeds_all):
    return pl.kernel(
        _seed_body,
        out_type=[jax.ShapeDtypeStruct((2 * NSEED_PAD, D), jnp.float32)],
        mesh=_sc_mesh,
        scratch_types=[
            pltpu.VMEM((SEEDS_PER_TILE,), jnp.int32),
            pltpu.VMEM((SEEDS_PER_TILE, D), jnp.float32),
            pltpu.SemaphoreType.DMA,
        ],
    )(ent_all, seeds_all)[0]


_BLK = 2048


def _dis(deg_ref):
    deg = deg_ref[:, 0:1] + 1.0
    return lax.rsqrt(deg)


def _mmraw_body(x_ref, w_ref, o_ref):
    o_ref[...] = jnp.dot(x_ref[...], w_ref[...],
                         preferred_element_type=jnp.float32)


def _scale_body(deg_ref, t_ref, o_ref):
    o_ref[...] = t_ref[...] * _dis(deg_ref)


def _mm1_body(deg_ref, s_ref, w_ref, o_ref):
    dis = _dis(deg_ref)
    h = jnp.maximum(s_ref[...] * dis, 0.0)
    o_ref[...] = jnp.dot(h, w_ref[...],
                         preferred_element_type=jnp.float32) * dis


def _fin_body(s_ref, o_ref):
    y = s_ref[...]
    nrm = jnp.sqrt(jnp.sum(y * y, axis=1, keepdims=True))
    o_ref[...] = y / jnp.maximum(nrm, 1e-12)


_row_spec = pl.BlockSpec((_BLK, D), lambda i: (i, 0))
_deg_spec = pl.BlockSpec((_BLK, DEGW), lambda i: (i, 0))
_w_spec = pl.BlockSpec((D, D), lambda i: (0, 0))
_out_struct = jax.ShapeDtypeStruct((2 * NP, D), jnp.float32)
_GRID = (2 * NP // _BLK,)


@jax.jit
def _mmraw_call(x, w):
    return pl.pallas_call(
        _mmraw_body,
        grid=_GRID,
        in_specs=[_row_spec, _w_spec],
        out_specs=_row_spec,
        out_shape=_out_struct,
    )(x, w)


@jax.jit
def _scale_call(deg16, t):
    return pl.pallas_call(
        _scale_body,
        grid=_GRID,
        in_specs=[_deg_spec, _row_spec],
        out_specs=_row_spec,
        out_shape=_out_struct,
    )(deg16, t)


@jax.jit
def _mm1_call(deg16, s, w):
    return pl.pallas_call(
        _mm1_body,
        grid=_GRID,
        in_specs=[_deg_spec, _row_spec, _w_spec],
        out_specs=_row_spec,
        out_shape=_out_struct,
    )(deg16, s, w)


@jax.jit
def _fin_call(s):
    return pl.pallas_call(
        _fin_body,
        grid=_GRID,
        in_specs=[_row_spec],
        out_specs=_row_spec,
        out_shape=_out_struct,
    )(s)


_SBLK = 1024


@jax.jit
def _fin_seed_call(rows):
    return pl.pallas_call(
        _fin_body,
        grid=(2 * NSEED_PAD // _SBLK,),
        in_specs=[pl.BlockSpec((_SBLK, D), lambda i: (i, 0))],
        out_specs=pl.BlockSpec((_SBLK, D), lambda i: (i, 0)),
        out_shape=jax.ShapeDtypeStruct((2 * NSEED_PAD, D), jnp.float32),
    )(rows)


@jax.jit
def kernel(feats_sr, feats_tg, W0, W1, edges_sr, edges_tg,
           sr_ent_seeds, tg_ent_seeds):
    dst_all = jnp.concatenate([edges_sr[:, 1], edges_tg[:, 1]])

    dead = N + (jnp.arange(PADE, dtype=jnp.int32) % (NP - N))
    src2d = jnp.concatenate(
        [edges_sr[:, 0], dead, edges_tg[:, 0] + NP, dead + NP]
    ).reshape(IDX_ROWS, PCHUNK)
    dst2d = jnp.concatenate(
        [edges_sr[:, 1], dead, edges_tg[:, 1], dead]
    ).reshape(IDX_ROWS, PCHUNK)

    pad = ((0, NP - N), (0, 0))
    feats_all = jnp.concatenate(
        [jnp.pad(feats_sr, pad), jnp.pad(feats_tg, pad)])

    deg_all = _deg_call(dst_all)

    t0 = _mmraw_call(feats_all, W0)
    u0 = _scale_call(deg_all, t0)
    s0 = _prop_call(u0, src2d, dst2d)
    u1 = _mm1_call(deg_all, s0, W1)
    s1 = _prop_call(u1, src2d, dst2d)
    ent_all = _fin_call(s1)

    spad = (0, NSEED_PAD - NSEED)
    seeds_all = jnp.concatenate(
        [jnp.pad(sr_ent_seeds.astype(jnp.int32), spad),
         jnp.pad(tg_ent_seeds.astype(jnp.int32), spad) + NP])
    seed_rows = _seed_call(s1, seeds_all)
    seed_all = _fin_seed_call(seed_rows)

    return (seed_all[:NSEED], seed_all[NSEED_PAD:NSEED_PAD + NSEED],
            ent_all[:N], ent_all[NP:NP + N])


# speedup vs baseline: 1.0097x; 1.0097x over previous
import jax
import jax.numpy as jnp
from jax import lax
from jax.experimental import pallas as pl
from jax.experimental.pallas import tpu as pltpu
from jax.experimental.pallas import tpu_sc as plsc

N = 10000
NP = 10240
D = 128
E = 320000
NSEED = 3000
NSEED_PAD = 3072

NC = 2
NS = 16

ROWS_PER_TILE = NP // NS
EDGES_PER_TILE = E // NS
CHUNK = 1000
NCHUNK = EDGES_PER_TILE // CHUNK
DEGW = 16
SEEDS_PER_TILE = 2 * NSEED_PAD // (NC * NS)

PCHUNK = 128
E_PAD = 327680
PADE = E_PAD - E
IDX_ROWS = 2 * E_PAD // PCHUNK
CROWS = IDX_ROWS // NC
TROWS = CROWS // NS
IB = 8
NBLK = TROWS // IB

_sc_mesh = plsc.VectorSubcoreMesh(core_axis_name="c", subcore_axis_name="s")


def _deg_body(dst_hbm, zeros_hbm, ones_hbm, out_hbm, acc, ones_v, idx_v):
    c = lax.axis_index("c")
    s = lax.axis_index("s")
    r0 = s * ROWS_PER_TILE

    pltpu.sync_copy(ones_hbm, ones_v)
    pltpu.sync_copy(zeros_hbm.at[pl.ds(r0, ROWS_PER_TILE)],
                    acc.at[pl.ds(r0, ROWS_PER_TILE)])
    plsc.subcore_barrier()

    def body(k, carry):
        base = pl.multiple_of(c * E + s * EDGES_PER_TILE + k * CHUNK, 8)
        pltpu.sync_copy(dst_hbm.at[pl.ds(base, CHUNK)], idx_v)
        pltpu.sync_copy(ones_v, acc.at[idx_v], add=True)
        return carry

    lax.fori_loop(0, NCHUNK, body, 0)
    plsc.subcore_barrier()
    pltpu.sync_copy(acc.at[pl.ds(r0, ROWS_PER_TILE)],
                    out_hbm.at[pl.ds(c * NP + r0, ROWS_PER_TILE)])


@jax.jit
def _deg_call(dst_all):
    zeros = jnp.zeros((NP, DEGW), jnp.float32)
    ones = jnp.ones((CHUNK, DEGW), jnp.float32)
    return pl.kernel(
        _deg_body,
        out_type=[jax.ShapeDtypeStruct((2 * NP, DEGW), jnp.float32)],
        mesh=_sc_mesh,
        compiler_params=pltpu.CompilerParams(use_tc_tiling_on_sc=False),
        scratch_types=[
            pltpu.VMEM_SHARED((NP, DEGW), jnp.float32),
            pltpu.VMEM((CHUNK, DEGW), jnp.float32),
            pltpu.VMEM((CHUNK,), jnp.int32),
        ],
    )(dst_all, zeros, ones)[0]


def _prop_body(u_hbm, src_hbm, dst_hbm, out_hbm,
               acc, rows_a, rows_b, src_i, dst_i, sem_a, sem_b,
               ssem_a, ssem_b):
    c = lax.axis_index("c")
    s = lax.axis_index("s")
    r0 = s * ROWS_PER_TILE

    pltpu.sync_copy(u_hbm.at[pl.ds(c * NP + r0, ROWS_PER_TILE)],
                    acc.at[pl.ds(r0, ROWS_PER_TILE)])
    plsc.subcore_barrier()

    rows = (rows_a, rows_b)
    sems = (sem_a, sem_b)
    ssems = (ssem_a, ssem_b)
    row0 = c * CROWS + s * TROWS

    def outer(k, carry):
        base = pl.multiple_of(row0 + k * IB, 8)
        pltpu.sync_copy(src_hbm.at[pl.ds(base, IB)], src_i)
        pltpu.sync_copy(dst_hbm.at[pl.ds(base, IB)], dst_i)
        g = [pltpu.async_copy(u_hbm.at[src_i.at[0]], rows[0], sems[0]),
             None]
        sc = [None, None]
        for j in range(IB):
            b = j % 2
            nb = (j + 1) % 2
            g[b].wait()
            sc[b] = pltpu.async_copy(rows[b], acc.at[dst_i.at[j]], ssems[b],
                                     add=True)
            if j + 1 < IB:
                if sc[nb] is not None:
                    sc[nb].wait()
                g[nb] = pltpu.async_copy(
                    u_hbm.at[src_i.at[j + 1]], rows[nb], sems[nb])
        sc[0].wait()
        sc[1].wait()
        return carry

    lax.fori_loop(0, NBLK, outer, 0)
    plsc.subcore_barrier()
    pltpu.sync_copy(acc.at[pl.ds(r0, ROWS_PER_TILE)],
                    out_hbm.at[pl.ds(c * NP + r0, ROWS_PER_TILE)])


@jax.jit
def _prop_call(u_all, src2d, dst2d):
    return pl.kernel(
        _prop_body,
        out_type=[jax.ShapeDtypeStruct((2 * NP, D), jnp.float32)],
        mesh=_sc_mesh,
        scratch_types=[
            pltpu.VMEM_SHARED((NP, D), jnp.float32),
            pltpu.VMEM((PCHUNK, D), jnp.float32),
            pltpu.VMEM((PCHUNK, D), jnp.float32),
            pltpu.VMEM((IB, PCHUNK), jnp.int32),
            pltpu.VMEM((IB, PCHUNK), jnp.int32),
            pltpu.SemaphoreType.DMA,
            pltpu.SemaphoreType.DMA,
            pltpu.SemaphoreType.DMA,
            pltpu.SemaphoreType.DMA,
        ],
    )(u_all, src2d, dst2d)[0]


def _seed_body(ent_hbm, seeds_hbm, out_hbm, idx_v, rows_v, gsem):
    c = lax.axis_index("c")
    s = lax.axis_index("s")
    base = (c * NS + s) * SEEDS_PER_TILE

    pltpu.sync_copy(seeds_hbm.at[pl.ds(base, SEEDS_PER_TILE)], idx_v)
    pltpu.async_copy(ent_hbm.at[idx_v], rows_v, gsem).wait()
    pltpu.sync_copy(rows_v, out_hbm.at[pl.ds(base, SEEDS_PER_TILE)])


@jax.jit
def _seed_call(ent_all, seeds_all):
    return pl.kernel(
        _seed_body,
        out_type=[jax.ShapeDtypeStruct((2 * NSEED_PAD, D), jnp.float32)],
        mesh=_sc_mesh,
        scratch_types=[
            pltpu.VMEM((SEEDS_PER_TILE,), jnp.int32),
            pltpu.VMEM((SEEDS_PER_TILE, D), jnp.float32),
            pltpu.SemaphoreType.DMA,
        ],
    )(ent_all, seeds_all)[0]


_BLK = 2048


def _dis(deg_ref):
    deg = deg_ref[:, 0:1] + 1.0
    return lax.rsqrt(deg)


def _mm0_body(deg_ref, x_ref, w_ref, o_ref):
    o_ref[...] = jnp.dot(x_ref[...], w_ref[...],
                         preferred_element_type=jnp.float32) * _dis(deg_ref)


def _mm1_body(deg_ref, s_ref, w_ref, o_ref):
    dis = _dis(deg_ref)
    h = jnp.maximum(s_ref[...] * dis, 0.0)
    o_ref[...] = jnp.dot(h, w_ref[...],
                         preferred_element_type=jnp.float32) * dis


def _fin_body(s_ref, o_ref):
    y = s_ref[...]
    nrm = jnp.sqrt(jnp.sum(y * y, axis=1, keepdims=True))
    o_ref[...] = y / jnp.maximum(nrm, 1e-12)


_row_spec = pl.BlockSpec((_BLK, D), lambda i: (i, 0))
_deg_spec = pl.BlockSpec((_BLK, DEGW), lambda i: (i, 0))
_w_spec = pl.BlockSpec((D, D), lambda i: (0, 0))
_out_struct = jax.ShapeDtypeStruct((2 * NP, D), jnp.float32)
_GRID = (2 * NP // _BLK,)


@jax.jit
def _mm0_call(deg16, x, w):
    return pl.pallas_call(
        _mm0_body,
        grid=_GRID,
        in_specs=[_deg_spec, _row_spec, _w_spec],
        out_specs=_row_spec,
        out_shape=_out_struct,
    )(deg16, x, w)


@jax.jit
def _mm1_call(deg16, s, w):
    return pl.pallas_call(
        _mm1_body,
        grid=_GRID,
        in_specs=[_deg_spec, _row_spec, _w_spec],
        out_specs=_row_spec,
        out_shape=_out_struct,
    )(deg16, s, w)


@jax.jit
def _fin_call(s):
    return pl.pallas_call(
        _fin_body,
        grid=_GRID,
        in_specs=[_row_spec],
        out_specs=_row_spec,
        out_shape=_out_struct,
    )(s)


@jax.jit
def kernel(feats_sr, feats_tg, W0, W1, edges_sr, edges_tg,
           sr_ent_seeds, tg_ent_seeds):
    dst_all = jnp.concatenate([edges_sr[:, 1], edges_tg[:, 1]])

    dead = N + (jnp.arange(PADE, dtype=jnp.int32) % (NP - N))
    src2d = jnp.concatenate(
        [edges_sr[:, 0], dead, edges_tg[:, 0] + NP, dead + NP]
    ).reshape(IDX_ROWS, PCHUNK)
    dst2d = jnp.concatenate(
        [edges_sr[:, 1], dead, edges_tg[:, 1], dead]
    ).reshape(IDX_ROWS, PCHUNK)

    pad = ((0, NP - N), (0, 0))
    feats_all = jnp.concatenate(
        [jnp.pad(feats_sr, pad), jnp.pad(feats_tg, pad)])

    deg_all = _deg_call(dst_all)

    u0 = _mm0_call(deg_all, feats_all, W0)
    s0 = _prop_call(u0, src2d, dst2d)
    u1 = _mm1_call(deg_all, s0, W1)
    s1 = _prop_call(u1, src2d, dst2d)
    ent_all = _fin_call(s1)

    spad = (0, NSEED_PAD - NSEED)
    seeds_all = jnp.concatenate(
        [jnp.pad(sr_ent_seeds.astype(jnp.int32), spad),
         jnp.pad(tg_ent_seeds.astype(jnp.int32), spad) + NP])
    seed_all = _seed_call(ent_all, seeds_all)

    return (seed_all[:NSEED], seed_all[NSEED_PAD:NSEED_PAD + NSEED],
            ent_all[:N], ent_all[NP:NP + N])


# speedup vs baseline: 1.0129x; 1.0032x over previous
import jax
import jax.numpy as jnp
from jax import lax
from jax.experimental import pallas as pl
from jax.experimental.pallas import tpu as pltpu
from jax.experimental.pallas import tpu_sc as plsc

N = 10000
NP = 10240
D = 128
E = 320000
NSEED = 3000
NSEED_PAD = 3072

NC = 2
NS = 16

ROWS_PER_TILE = NP // NS
EDGES_PER_TILE = E // NS
CHUNK = 2000
NCHUNK = EDGES_PER_TILE // CHUNK
DEGW = 16
SEEDS_PER_TILE = 2 * NSEED_PAD // (NC * NS)

PCHUNK = 128
E_PAD = 327680
PADE = E_PAD - E
IDX_ROWS = 2 * E_PAD // PCHUNK
CROWS = IDX_ROWS // NC
TROWS = CROWS // NS
IB = 8
NBLK = TROWS // IB

_sc_mesh = plsc.VectorSubcoreMesh(core_axis_name="c", subcore_axis_name="s")


def _deg_body(dst_hbm, zeros_hbm, ones_hbm, out_hbm, acc, ones_v, idx_v):
    c = lax.axis_index("c")
    s = lax.axis_index("s")
    r0 = s * ROWS_PER_TILE

    pltpu.sync_copy(ones_hbm, ones_v)
    pltpu.sync_copy(zeros_hbm.at[pl.ds(r0, ROWS_PER_TILE)],
                    acc.at[pl.ds(r0, ROWS_PER_TILE)])
    plsc.subcore_barrier()

    def body(k, carry):
        base = pl.multiple_of(c * E + s * EDGES_PER_TILE + k * CHUNK, 8)
        pltpu.sync_copy(dst_hbm.at[pl.ds(base, CHUNK)], idx_v)
        pltpu.sync_copy(ones_v, acc.at[idx_v], add=True)
        return carry

    lax.fori_loop(0, NCHUNK, body, 0)
    plsc.subcore_barrier()
    pltpu.sync_copy(acc.at[pl.ds(r0, ROWS_PER_TILE)],
                    out_hbm.at[pl.ds(c * NP + r0, ROWS_PER_TILE)])


@jax.jit
def _deg_call(dst_all):
    zeros = jnp.zeros((NP, DEGW), jnp.float32)
    ones = jnp.ones((CHUNK, DEGW), jnp.float32)
    return pl.kernel(
        _deg_body,
        out_type=[jax.ShapeDtypeStruct((2 * NP, DEGW), jnp.float32)],
        mesh=_sc_mesh,
        compiler_params=pltpu.CompilerParams(use_tc_tiling_on_sc=False),
        scratch_types=[
            pltpu.VMEM_SHARED((NP, DEGW), jnp.float32),
            pltpu.VMEM((CHUNK, DEGW), jnp.float32),
            pltpu.VMEM((CHUNK,), jnp.int32),
        ],
    )(dst_all, zeros, ones)[0]


def _prop_body(u_hbm, src_hbm, dst_hbm, out_hbm,
               acc, rows_a, rows_b, src_i, dst_i, sem_a, sem_b,
               ssem_a, ssem_b):
    c = lax.axis_index("c")
    s = lax.axis_index("s")
    r0 = s * ROWS_PER_TILE

    pltpu.sync_copy(u_hbm.at[pl.ds(c * NP + r0, ROWS_PER_TILE)],
                    acc.at[pl.ds(r0, ROWS_PER_TILE)])
    plsc.subcore_barrier()

    rows = (rows_a, rows_b)
    sems = (sem_a, sem_b)
    ssems = (ssem_a, ssem_b)
    row0 = c * CROWS + s * TROWS

    def outer(k, carry):
        base = pl.multiple_of(row0 + k * IB, 8)
        pltpu.sync_copy(src_hbm.at[pl.ds(base, IB)], src_i)
        pltpu.sync_copy(dst_hbm.at[pl.ds(base, IB)], dst_i)
        g = [pltpu.async_copy(u_hbm.at[src_i.at[0]], rows[0], sems[0]),
             None]
        sc = [None, None]
        for j in range(IB):
            b = j % 2
            nb = (j + 1) % 2
            g[b].wait()
            sc[b] = pltpu.async_copy(rows[b], acc.at[dst_i.at[j]], ssems[b],
                                     add=True)
            if j + 1 < IB:
                if sc[nb] is not None:
                    sc[nb].wait()
                g[nb] = pltpu.async_copy(
                    u_hbm.at[src_i.at[j + 1]], rows[nb], sems[nb])
        sc[0].wait()
        sc[1].wait()
        return carry

    lax.fori_loop(0, NBLK, outer, 0)
    plsc.subcore_barrier()
    pltpu.sync_copy(acc.at[pl.ds(r0, ROWS_PER_TILE)],
                    out_hbm.at[pl.ds(c * NP + r0, ROWS_PER_TILE)])


@jax.jit
def _prop_call(u_all, src2d, dst2d):
    return pl.kernel(
        _prop_body,
        out_type=[jax.ShapeDtypeStruct((2 * NP, D), jnp.float32)],
        mesh=_sc_mesh,
        scratch_types=[
            pltpu.VMEM_SHARED((NP, D), jnp.float32),
            pltpu.VMEM((PCHUNK, D), jnp.float32),
            pltpu.VMEM((PCHUNK, D), jnp.float32),
            pltpu.VMEM((IB, PCHUNK), jnp.int32),
            pltpu.VMEM((IB, PCHUNK), jnp.int32),
            pltpu.SemaphoreType.DMA,
            pltpu.SemaphoreType.DMA,
            pltpu.SemaphoreType.DMA,
            pltpu.SemaphoreType.DMA,
        ],
    )(u_all, src2d, dst2d)[0]


def _seed_body(ent_hbm, seeds_hbm, out_hbm, idx_v, rows_v, gsem):
    c = lax.axis_index("c")
    s = lax.axis_index("s")
    base = (c * NS + s) * SEEDS_PER_TILE

    pltpu.sync_copy(seeds_hbm.at[pl.ds(base, SEEDS_PER_TILE)], idx_v)
    pltpu.async_copy(ent_hbm.at[idx_v], rows_v, gsem).wait()
    pltpu.sync_copy(rows_v, out_hbm.at[pl.ds(base, SEEDS_PER_TILE)])


@jax.jit
def _seed_call(ent_all, seeds_all):
    return pl.kernel(
        _seed_body,
        out_type=[jax.ShapeDtypeStruct((2 * NSEED_PAD, D), jnp.float32)],
        mesh=_sc_mesh,
        scratch_types=[
            pltpu.VMEM((SEEDS_PER_TILE,), jnp.int32),
            pltpu.VMEM((SEEDS_PER_TILE, D), jnp.float32),
            pltpu.SemaphoreType.DMA,
        ],
    )(ent_all, seeds_all)[0]


_BLK = 2048


def _dis(deg_ref):
    deg = deg_ref[:, 0:1] + 1.0
    return lax.rsqrt(deg)


def _mm0_body(deg_ref, x_ref, w_ref, o_ref):
    o_ref[...] = jnp.dot(x_ref[...], w_ref[...],
                         preferred_element_type=jnp.float32) * _dis(deg_ref)


def _mm1_body(deg_ref, s_ref, w_ref, o_ref):
    dis = _dis(deg_ref)
    h = jnp.maximum(s_ref[...] * dis, 0.0)
    o_ref[...] = jnp.dot(h, w_ref[...],
                         preferred_element_type=jnp.float32) * dis


def _fin_body(s_ref, o_ref):
    y = s_ref[...]
    nrm = jnp.sqrt(jnp.sum(y * y, axis=1, keepdims=True))
    o_ref[...] = y / jnp.maximum(nrm, 1e-12)


_row_spec = pl.BlockSpec((_BLK, D), lambda i: (i, 0))
_deg_spec = pl.BlockSpec((_BLK, DEGW), lambda i: (i, 0))
_w_spec = pl.BlockSpec((D, D), lambda i: (0, 0))
_out_struct = jax.ShapeDtypeStruct((2 * NP, D), jnp.float32)
_GRID = (2 * NP // _BLK,)


@jax.jit
def _mm0_call(deg16, x, w):
    return pl.pallas_call(
        _mm0_body,
        grid=_GRID,
        in_specs=[_deg_spec, _row_spec, _w_spec],
        out_specs=_row_spec,
        out_shape=_out_struct,
    )(deg16, x, w)


@jax.jit
def _mm1_call(deg16, s, w):
    return pl.pallas_call(
        _mm1_body,
        grid=_GRID,
        in_specs=[_deg_spec, _row_spec, _w_spec],
        out_specs=_row_spec,
        out_shape=_out_struct,
    )(deg16, s, w)


@jax.jit
def _fin_call(s):
    return pl.pallas_call(
        _fin_body,
        grid=_GRID,
        in_specs=[_row_spec],
        out_specs=_row_spec,
        out_shape=_out_struct,
    )(s)


@jax.jit
def kernel(feats_sr, feats_tg, W0, W1, edges_sr, edges_tg,
           sr_ent_seeds, tg_ent_seeds):
    dst_all = jnp.concatenate([edges_sr[:, 1], edges_tg[:, 1]])

    dead = N + (jnp.arange(PADE, dtype=jnp.int32) % (NP - N))
    src2d = jnp.concatenate(
        [edges_sr[:, 0], dead, edges_tg[:, 0] + NP, dead + NP]
    ).reshape(IDX_ROWS, PCHUNK)
    dst2d = jnp.concatenate(
        [edges_sr[:, 1], dead, edges_tg[:, 1], dead]
    ).reshape(IDX_ROWS, PCHUNK)

    pad = ((0, NP - N), (0, 0))
    feats_all = jnp.concatenate(
        [jnp.pad(feats_sr, pad), jnp.pad(feats_tg, pad)])

    deg_all = _deg_call(dst_all)

    u0 = _mm0_call(deg_all, feats_all, W0)
    s0 = _prop_call(u0, src2d, dst2d)
    u1 = _mm1_call(deg_all, s0, W1)
    s1 = _prop_call(u1, src2d, dst2d)
    ent_all = _fin_call(s1)

    spad = (0, NSEED_PAD - NSEED)
    seeds_all = jnp.concatenate(
        [jnp.pad(sr_ent_seeds.astype(jnp.int32), spad),
         jnp.pad(tg_ent_seeds.astype(jnp.int32), spad) + NP])
    seed_all = _seed_call(ent_all, seeds_all)

    return (seed_all[:NSEED], seed_all[NSEED_PAD:NSEED_PAD + NSEED],
            ent_all[:N], ent_all[NP:NP + N])


# speedup vs baseline: 1.0379x; 1.0247x over previous
import jax
import jax.numpy as jnp
from jax import lax
from jax.experimental import pallas as pl
from jax.experimental.pallas import tpu as pltpu
from jax.experimental.pallas import tpu_sc as plsc

N = 10000
NP = 10240
D = 128
E = 320000
NSEED = 3000
NSEED_PAD = 3072

NC = 2
NS = 16

ROWS_PER_TILE = NP // NS
EDGES_PER_TILE = E // NS
CHUNK = 2000
NCHUNK = EDGES_PER_TILE // CHUNK
DEGW = 16
SEEDS_PER_TILE = 2 * NSEED_PAD // (NC * NS)

PCHUNK = 128
E_PAD = 327680
PADE = E_PAD - E
IDX_ROWS = 2 * E_PAD // PCHUNK
CROWS = IDX_ROWS // NC
TROWS = CROWS // NS
IB = 8
NBLK = TROWS // IB
TOTBLK = IDX_ROWS // IB
BLK_PER_TILE = NBLK

_sc_mesh = plsc.VectorSubcoreMesh(core_axis_name="c", subcore_axis_name="s")


def _deg_body(dst_hbm, zeros_hbm, ones_hbm, out_hbm, acc, ones_v, idx_v):
    c = lax.axis_index("c")
    s = lax.axis_index("s")
    r0 = s * ROWS_PER_TILE

    pltpu.sync_copy(ones_hbm, ones_v)
    pltpu.sync_copy(zeros_hbm.at[pl.ds(r0, ROWS_PER_TILE)],
                    acc.at[pl.ds(r0, ROWS_PER_TILE)])
    plsc.subcore_barrier()

    def body(k, carry):
        base = pl.multiple_of(c * E + s * EDGES_PER_TILE + k * CHUNK, 8)
        pltpu.sync_copy(dst_hbm.at[pl.ds(base, CHUNK)], idx_v)
        pltpu.sync_copy(ones_v, acc.at[idx_v], add=True)
        return carry

    lax.fori_loop(0, NCHUNK, body, 0)
    plsc.subcore_barrier()
    pltpu.sync_copy(acc.at[pl.ds(r0, ROWS_PER_TILE)],
                    out_hbm.at[pl.ds(c * NP + r0, ROWS_PER_TILE)])


@jax.jit
def _deg_call(dst_all):
    zeros = jnp.zeros((NP, DEGW), jnp.float32)
    ones = jnp.ones((CHUNK, DEGW), jnp.float32)
    return pl.kernel(
        _deg_body,
        out_type=[jax.ShapeDtypeStruct((2 * NP, DEGW), jnp.float32)],
        mesh=_sc_mesh,
        compiler_params=pltpu.CompilerParams(use_tc_tiling_on_sc=False),
        scratch_types=[
            pltpu.VMEM_SHARED((NP, DEGW), jnp.float32),
            pltpu.VMEM((CHUNK, DEGW), jnp.float32),
            pltpu.VMEM((CHUNK,), jnp.int32),
        ],
    )(dst_all, zeros, ones)[0]


def _prop_body(u_hbm, idx_hbm, out_hbm,
               acc, rows_a, rows_b, idx_v, sem_a, sem_b,
               ssem_a, ssem_b):
    c = lax.axis_index("c")
    s = lax.axis_index("s")
    r0 = s * ROWS_PER_TILE

    pltpu.sync_copy(u_hbm.at[pl.ds(c * NP + r0, ROWS_PER_TILE)],
                    acc.at[pl.ds(r0, ROWS_PER_TILE)])
    plsc.subcore_barrier()

    rows = (rows_a, rows_b)
    sems = (sem_a, sem_b)
    ssems = (ssem_a, ssem_b)
    blk0 = c * (NS * BLK_PER_TILE) + s * BLK_PER_TILE

    def outer(k, carry):
        base = pl.multiple_of((blk0 + k) * (2 * IB), 8)
        pltpu.sync_copy(idx_hbm.at[pl.ds(base, 2 * IB)], idx_v)
        g = [pltpu.async_copy(u_hbm.at[idx_v.at[0]], rows[0], sems[0]),
             None]
        sc = [None, None]
        for j in range(IB):
            b = j % 2
            nb = (j + 1) % 2
            g[b].wait()
            sc[b] = pltpu.async_copy(rows[b], acc.at[idx_v.at[IB + j]],
                                     ssems[b], add=True)
            if j + 1 < IB:
                if sc[nb] is not None:
                    sc[nb].wait()
                g[nb] = pltpu.async_copy(
                    u_hbm.at[idx_v.at[j + 1]], rows[nb], sems[nb])
        sc[0].wait()
        sc[1].wait()
        return carry

    lax.fori_loop(0, NBLK, outer, 0)
    plsc.subcore_barrier()
    pltpu.sync_copy(acc.at[pl.ds(r0, ROWS_PER_TILE)],
                    out_hbm.at[pl.ds(c * NP + r0, ROWS_PER_TILE)])


@jax.jit
def _prop_call(u_all, idx_comb):
    return pl.kernel(
        _prop_body,
        out_type=[jax.ShapeDtypeStruct((2 * NP, D), jnp.float32)],
        mesh=_sc_mesh,
        scratch_types=[
            pltpu.VMEM_SHARED((NP, D), jnp.float32),
            pltpu.VMEM((PCHUNK, D), jnp.float32),
            pltpu.VMEM((PCHUNK, D), jnp.float32),
            pltpu.VMEM((2 * IB, PCHUNK), jnp.int32),
            pltpu.SemaphoreType.DMA,
            pltpu.SemaphoreType.DMA,
            pltpu.SemaphoreType.DMA,
            pltpu.SemaphoreType.DMA,
        ],
    )(u_all, idx_comb)[0]


def _seed_body(ent_hbm, seeds_hbm, out_hbm, idx_v, rows_v, gsem):
    c = lax.axis_index("c")
    s = lax.axis_index("s")
    base = (c * NS + s) * SEEDS_PER_TILE

    pltpu.sync_copy(seeds_hbm.at[pl.ds(base, SEEDS_PER_TILE)], idx_v)
    pltpu.async_copy(ent_hbm.at[idx_v], rows_v, gsem).wait()
    pltpu.sync_copy(rows_v, out_hbm.at[pl.ds(base, SEEDS_PER_TILE)])


@jax.jit
def _seed_call(ent_all, seeds_all):
    return pl.kernel(
        _seed_body,
        out_type=[jax.ShapeDtypeStruct((2 * NSEED_PAD, D), jnp.float32)],
        mesh=_sc_mesh,
        scratch_types=[
            pltpu.VMEM((SEEDS_PER_TILE,), jnp.int32),
            pltpu.VMEM((SEEDS_PER_TILE, D), jnp.float32),
            pltpu.SemaphoreType.DMA,
        ],
    )(ent_all, seeds_all)[0]


_BLK = 2048


def _dis(deg_ref):
    deg = deg_ref[:, 0:1] + 1.0
    return lax.rsqrt(deg)


def _mm0_body(deg_ref, x_ref, w_ref, o_ref):
    o_ref[...] = jnp.dot(x_ref[...], w_ref[...],
                         preferred_element_type=jnp.float32) * _dis(deg_ref)


def _mm1_body(deg_ref, s_ref, w_ref, o_ref):
    dis = _dis(deg_ref)
    h = jnp.maximum(s_ref[...] * dis, 0.0)
    o_ref[...] = jnp.dot(h, w_ref[...],
                         preferred_element_type=jnp.float32) * dis


def _fin_body(s_ref, o_ref):
    y = s_ref[...]
    nrm = jnp.sqrt(jnp.sum(y * y, axis=1, keepdims=True))
    o_ref[...] = y / jnp.maximum(nrm, 1e-12)


_row_spec = pl.BlockSpec((_BLK, D), lambda i: (i, 0))
_deg_spec = pl.BlockSpec((_BLK, DEGW), lambda i: (i, 0))
_w_spec = pl.BlockSpec((D, D), lambda i: (0, 0))
_out_struct = jax.ShapeDtypeStruct((2 * NP, D), jnp.float32)
_GRID = (2 * NP // _BLK,)


@jax.jit
def _mm0_call(deg16, x, w):
    return pl.pallas_call(
        _mm0_body,
        grid=_GRID,
        in_specs=[_deg_spec, _row_spec, _w_spec],
        out_specs=_row_spec,
        out_shape=_out_struct,
    )(deg16, x, w)


@jax.jit
def _mm1_call(deg16, s, w):
    return pl.pallas_call(
        _mm1_body,
        grid=_GRID,
        in_specs=[_deg_spec, _row_spec, _w_spec],
        out_specs=_row_spec,
        out_shape=_out_struct,
    )(deg16, s, w)


@jax.jit
def _fin_call(s):
    return pl.pallas_call(
        _fin_body,
        grid=_GRID,
        in_specs=[_row_spec],
        out_specs=_row_spec,
        out_shape=_out_struct,
    )(s)


@jax.jit
def kernel(feats_sr, feats_tg, W0, W1, edges_sr, edges_tg,
           sr_ent_seeds, tg_ent_seeds):
    dst_all = jnp.concatenate([edges_sr[:, 1], edges_tg[:, 1]])

    dead = N + (jnp.arange(PADE, dtype=jnp.int32) % (NP - N))
    src3 = jnp.concatenate(
        [edges_sr[:, 0], dead, edges_tg[:, 0] + NP, dead + NP]
    ).reshape(TOTBLK, IB, PCHUNK)
    dst3 = jnp.concatenate(
        [edges_sr[:, 1], dead, edges_tg[:, 1], dead]
    ).reshape(TOTBLK, IB, PCHUNK)
    idx_comb = jnp.concatenate([src3, dst3], axis=1).reshape(
        TOTBLK * 2 * IB, PCHUNK)

    pad = ((0, NP - N), (0, 0))
    feats_all = jnp.concatenate(
        [jnp.pad(feats_sr, pad), jnp.pad(feats_tg, pad)])

    deg_all = _deg_call(dst_all)

    u0 = _mm0_call(deg_all, feats_all, W0)
    s0 = _prop_call(u0, idx_comb)
    u1 = _mm1_call(deg_all, s0, W1)
    s1 = _prop_call(u1, idx_comb)
    ent_all = _fin_call(s1)

    spad = (0, NSEED_PAD - NSEED)
    seeds_all = jnp.concatenate(
        [jnp.pad(sr_ent_seeds.astype(jnp.int32), spad),
         jnp.pad(tg_ent_seeds.astype(jnp.int32), spad) + NP])
    seed_all = _seed_call(ent_all, seeds_all)

    return (seed_all[:NSEED], seed_all[NSEED_PAD:NSEED_PAD + NSEED],
            ent_all[:N], ent_all[NP:NP + N])


# speedup vs baseline: 1.0409x; 1.0029x over previous
import jax
import jax.numpy as jnp
from jax import lax
from jax.experimental import pallas as pl
from jax.experimental.pallas import tpu as pltpu
from jax.experimental.pallas import tpu_sc as plsc

N = 10000
NP = 10240
D = 128
E = 320000
NSEED = 3000
NSEED_PAD = 3072

NC = 2
NS = 16

ROWS_PER_TILE = NP // NS
EDGES_PER_TILE = E // NS
CHUNK = 2000
NCHUNK = EDGES_PER_TILE // CHUNK
DEGW = 16
SEEDS_PER_TILE = 2 * NSEED_PAD // (NC * NS)

PCHUNK = 128
E_PAD = 327680
PADE = E_PAD - E
IDX_ROWS = 2 * E_PAD // PCHUNK
CROWS = IDX_ROWS // NC
TROWS = CROWS // NS
IB = 8
NBLK = TROWS // IB
TOTBLK = IDX_ROWS // IB
BLK_PER_TILE = NBLK

_sc_mesh = plsc.VectorSubcoreMesh(core_axis_name="c", subcore_axis_name="s")


def _deg_body(dst_hbm, zeros_hbm, ones_hbm, out_hbm, acc, ones_v, idx_v):
    c = lax.axis_index("c")
    s = lax.axis_index("s")
    r0 = s * ROWS_PER_TILE

    pltpu.sync_copy(ones_hbm, ones_v)
    pltpu.sync_copy(zeros_hbm.at[pl.ds(r0, ROWS_PER_TILE)],
                    acc.at[pl.ds(r0, ROWS_PER_TILE)])
    plsc.subcore_barrier()

    def body(k, carry):
        base = pl.multiple_of(c * E + s * EDGES_PER_TILE + k * CHUNK, 8)
        pltpu.sync_copy(dst_hbm.at[pl.ds(base, CHUNK)], idx_v)
        pltpu.sync_copy(ones_v, acc.at[idx_v], add=True)
        return carry

    lax.fori_loop(0, NCHUNK, body, 0)
    plsc.subcore_barrier()
    pltpu.sync_copy(acc.at[pl.ds(r0, ROWS_PER_TILE)],
                    out_hbm.at[pl.ds(c * NP + r0, ROWS_PER_TILE)])


@jax.jit
def _deg_call(dst_all):
    zeros = jnp.zeros((NP, DEGW), jnp.float32)
    ones = jnp.ones((CHUNK, DEGW), jnp.float32)
    return pl.kernel(
        _deg_body,
        out_type=[jax.ShapeDtypeStruct((2 * NP, DEGW), jnp.float32)],
        mesh=_sc_mesh,
        compiler_params=pltpu.CompilerParams(use_tc_tiling_on_sc=False),
        scratch_types=[
            pltpu.VMEM_SHARED((NP, DEGW), jnp.float32),
            pltpu.VMEM((CHUNK, DEGW), jnp.float32),
            pltpu.VMEM((CHUNK,), jnp.int32),
        ],
    )(dst_all, zeros, ones)[0]


def _prop_body(u_hbm, idx_hbm, out_hbm,
               acc, rows_a, rows_b, idx_v, sem_a, sem_b,
               ssem_a, ssem_b, isem):
    c = lax.axis_index("c")
    s = lax.axis_index("s")
    r0 = s * ROWS_PER_TILE

    blk0 = c * (NS * BLK_PER_TILE) + s * BLK_PER_TILE
    base0 = pl.multiple_of(blk0 * (2 * IB), 8)
    pltpu.async_copy(idx_hbm.at[pl.ds(base0, 2 * IB)], idx_v, isem)

    pltpu.sync_copy(u_hbm.at[pl.ds(c * NP + r0, ROWS_PER_TILE)],
                    acc.at[pl.ds(r0, ROWS_PER_TILE)])
    plsc.subcore_barrier()

    rows = (rows_a, rows_b)
    sems = (sem_a, sem_b)
    ssems = (ssem_a, ssem_b)
    max_base = (TOTBLK - 1) * (2 * IB)

    def outer(k, carry):
        base = pl.multiple_of((blk0 + k) * (2 * IB), 8)
        pltpu.make_async_copy(
            idx_hbm.at[pl.ds(base, 2 * IB)], idx_v, isem).wait()
        g = [pltpu.async_copy(u_hbm.at[idx_v.at[0]], rows[0], sems[0]),
             None]
        sc = [None, None]
        for j in range(IB):
            b = j % 2
            nb = (j + 1) % 2
            g[b].wait()
            sc[b] = pltpu.async_copy(rows[b], acc.at[idx_v.at[IB + j]],
                                     ssems[b], add=True)
            if j + 1 < IB:
                if sc[nb] is not None:
                    sc[nb].wait()
                g[nb] = pltpu.async_copy(
                    u_hbm.at[idx_v.at[j + 1]], rows[nb], sems[nb])
        sc[0].wait()
        sc[1].wait()
        nbase = pl.multiple_of(
            jnp.minimum((blk0 + k + 1) * (2 * IB), max_base), 8)
        pltpu.async_copy(idx_hbm.at[pl.ds(nbase, 2 * IB)], idx_v, isem)
        return carry

    lax.fori_loop(0, NBLK, outer, 0)
    pltpu.make_async_copy(
        idx_hbm.at[pl.ds(pl.multiple_of(max_base, 8), 2 * IB)],
        idx_v, isem).wait()
    plsc.subcore_barrier()
    pltpu.sync_copy(acc.at[pl.ds(r0, ROWS_PER_TILE)],
                    out_hbm.at[pl.ds(c * NP + r0, ROWS_PER_TILE)])


@jax.jit
def _prop_call(u_all, idx_comb):
    return pl.kernel(
        _prop_body,
        out_type=[jax.ShapeDtypeStruct((2 * NP, D), jnp.float32)],
        mesh=_sc_mesh,
        scratch_types=[
            pltpu.VMEM_SHARED((NP, D), jnp.float32),
            pltpu.VMEM((PCHUNK, D), jnp.float32),
            pltpu.VMEM((PCHUNK, D), jnp.float32),
            pltpu.VMEM((2 * IB, PCHUNK), jnp.int32),
            pltpu.SemaphoreType.DMA,
            pltpu.SemaphoreType.DMA,
            pltpu.SemaphoreType.DMA,
            pltpu.SemaphoreType.DMA,
            pltpu.SemaphoreType.DMA,
        ],
    )(u_all, idx_comb)[0]


def _seed_body(ent_hbm, seeds_hbm, out_hbm, idx_v, rows_v, gsem):
    c = lax.axis_index("c")
    s = lax.axis_index("s")
    base = (c * NS + s) * SEEDS_PER_TILE

    pltpu.sync_copy(seeds_hbm.at[pl.ds(base, SEEDS_PER_TILE)], idx_v)
    pltpu.async_copy(ent_hbm.at[idx_v], rows_v, gsem).wait()
    pltpu.sync_copy(rows_v, out_hbm.at[pl.ds(base, SEEDS_PER_TILE)])


@jax.jit
def _seed_call(ent_all, seeds_all):
    return pl.kernel(
        _seed_body,
        out_type=[jax.ShapeDtypeStruct((2 * NSEED_PAD, D), jnp.float32)],
        mesh=_sc_mesh,
        scratch_types=[
            pltpu.VMEM((SEEDS_PER_TILE,), jnp.int32),
            pltpu.VMEM((SEEDS_PER_TILE, D), jnp.float32),
            pltpu.SemaphoreType.DMA,
        ],
    )(ent_all, seeds_all)[0]


_BLK = 2048


def _dis(deg_ref):
    deg = deg_ref[:, 0:1] + 1.0
    return lax.rsqrt(deg)


def _mm0_body(deg_ref, x_ref, w_ref, o_ref):
    o_ref[...] = jnp.dot(x_ref[...], w_ref[...],
                         preferred_element_type=jnp.float32) * _dis(deg_ref)


def _mm1_body(deg_ref, s_ref, w_ref, o_ref):
    dis = _dis(deg_ref)
    h = jnp.maximum(s_ref[...] * dis, 0.0)
    o_ref[...] = jnp.dot(h, w_ref[...],
                         preferred_element_type=jnp.float32) * dis


def _fin_body(s_ref, o_ref):
    y = s_ref[...]
    nrm = jnp.sqrt(jnp.sum(y * y, axis=1, keepdims=True))
    o_ref[...] = y / jnp.maximum(nrm, 1e-12)


_row_spec = pl.BlockSpec((_BLK, D), lambda i: (i, 0))
_deg_spec = pl.BlockSpec((_BLK, DEGW), lambda i: (i, 0))
_w_spec = pl.BlockSpec((D, D), lambda i: (0, 0))
_out_struct = jax.ShapeDtypeStruct((2 * NP, D), jnp.float32)
_GRID = (2 * NP // _BLK,)


@jax.jit
def _mm0_call(deg16, x, w):
    return pl.pallas_call(
        _mm0_body,
        grid=_GRID,
        in_specs=[_deg_spec, _row_spec, _w_spec],
        out_specs=_row_spec,
        out_shape=_out_struct,
    )(deg16, x, w)


@jax.jit
def _mm1_call(deg16, s, w):
    return pl.pallas_call(
        _mm1_body,
        grid=_GRID,
        in_specs=[_deg_spec, _row_spec, _w_spec],
        out_specs=_row_spec,
        out_shape=_out_struct,
    )(deg16, s, w)


@jax.jit
def _fin_call(s):
    return pl.pallas_call(
        _fin_body,
        grid=_GRID,
        in_specs=[_row_spec],
        out_specs=_row_spec,
        out_shape=_out_struct,
    )(s)


@jax.jit
def kernel(feats_sr, feats_tg, W0, W1, edges_sr, edges_tg,
           sr_ent_seeds, tg_ent_seeds):
    dst_all = jnp.concatenate([edges_sr[:, 1], edges_tg[:, 1]])

    dead = N + (jnp.arange(PADE, dtype=jnp.int32) % (NP - N))
    src3 = jnp.concatenate(
        [edges_sr[:, 0], dead, edges_tg[:, 0] + NP, dead + NP]
    ).reshape(TOTBLK, IB, PCHUNK)
    dst3 = jnp.concatenate(
        [edges_sr[:, 1], dead, edges_tg[:, 1], dead]
    ).reshape(TOTBLK, IB, PCHUNK)
    idx_comb = jnp.concatenate([src3, dst3], axis=1).reshape(
        TOTBLK * 2 * IB, PCHUNK)

    pad = ((0, NP - N), (0, 0))
    feats_all = jnp.concatenate(
        [jnp.pad(feats_sr, pad), jnp.pad(feats_tg, pad)])

    deg_all = _deg_call(dst_all)

    u0 = _mm0_call(deg_all, feats_all, W0)
    s0 = _prop_call(u0, idx_comb)
    u1 = _mm1_call(deg_all, s0, W1)
    s1 = _prop_call(u1, idx_comb)
    ent_all = _fin_call(s1)

    spad = (0, NSEED_PAD - NSEED)
    seeds_all = jnp.concatenate(
        [jnp.pad(sr_ent_seeds.astype(jnp.int32), spad),
         jnp.pad(tg_ent_seeds.astype(jnp.int32), spad) + NP])
    seed_all = _seed_call(ent_all, seeds_all)

    return (seed_all[:NSEED], seed_all[NSEED_PAD:NSEED_PAD + NSEED],
            ent_all[:N], ent_all[NP:NP + N])
